# baseline (speedup 1.0000x reference)
import functools

import jax
import jax.numpy as jnp
from jax import lax
from jax.experimental import pallas as pl
from jax.experimental.pallas import tpu as pltpu
from jax.experimental.pallas import tpu_sc as plsc

B = 16384
D = 128
H = 512
OUT = 3

_info = plsc.get_sparse_core_info()
_NC, _NS = _info.num_cores, _info.num_subcores
_NW = _NC * _NS


def _sc_gather(table, idx2d, nb):
    ch = (nb // 128) // _NW
    mesh = plsc.VectorSubcoreMesh(core_axis_name="c", subcore_axis_name="s")

    @functools.partial(
        pl.kernel,
        mesh=mesh,
        out_type=jax.ShapeDtypeStruct((nb, D), jnp.float32),
        scratch_types=[
            pltpu.VMEM((ch, 128), jnp.int32),
            pltpu.VMEM((ch, 128, D), jnp.float32),
            pltpu.SemaphoreType.DMA,
            pltpu.SemaphoreType.DMA,
        ],
    )
    def k(table_hbm, idx_hbm, out_hbm, idx_v, rows_v, gsem, osem):
        wid = lax.axis_index("s") * _NC + lax.axis_index("c")
        base = wid * ch
        pltpu.sync_copy(idx_hbm.at[pl.ds(base, ch)], idx_v)
        gathers = [
            pltpu.async_copy(table_hbm.at[idx_v.at[j]], rows_v.at[j], gsem)
            for j in range(ch)
        ]
        out_copies = []
        for j in range(ch):
            gathers[j].wait()
            out_copies.append(
                pltpu.async_copy(
                    rows_v.at[j], out_hbm.at[pl.ds((base + j) * 128, 128)], osem
                )
            )
        for c in out_copies:
            c.wait()

    return k(table, idx2d)


def _tc_mlp(x, w1t, b1r, w2, b2c, nb):
    blk = 2048
    nbuf = 4
    grid = nb // blk

    def xcopy(x_hbm, xbuf, sems, block, slot):
        return pltpu.make_async_copy(
            x_hbm.at[pl.ds(block * blk, blk), :], xbuf.at[slot], sems.at[slot]
        )

    def body(x_hbm, w1_ref, b1_ref, w2_ref, b2_ref, o_ref, xbuf, sems):
        i = pl.program_id(0)
        slot = lax.rem(i, nbuf)

        @pl.when(i == 0)
        def _():
            for s in range(nbuf - 1):
                xcopy(x_hbm, xbuf, sems, s, s).start()

        nxt = i + nbuf - 1

        @pl.when(nxt < grid)
        def _():
            xcopy(x_hbm, xbuf, sems, nxt, lax.rem(nxt, nbuf)).start()

        xcopy(x_hbm, xbuf, sems, i, slot).wait()
        xb = xbuf[slot]
        h = jnp.dot(xb, w1_ref[:], preferred_element_type=jnp.float32)
        h = jnp.maximum(h + b1_ref[:], 0.0)
        ot = lax.dot_general(
            w2_ref[:], h, (((1,), (1,)), ((), ())),
            preferred_element_type=jnp.float32,
        )
        o_ref[:] = ot + b2_ref[:]

    return pl.pallas_call(
        body,
        grid=(grid,),
        in_specs=[
            pl.BlockSpec(memory_space=pl.ANY),
            pl.BlockSpec((D, H), lambda i: (0, 0)),
            pl.BlockSpec((1, H), lambda i: (0, 0)),
            pl.BlockSpec((OUT, H), lambda i: (0, 0)),
            pl.BlockSpec((OUT, 1), lambda i: (0, 0)),
        ],
        out_specs=pl.BlockSpec((OUT, blk), lambda i: (0, i)),
        out_shape=jax.ShapeDtypeStruct((OUT, nb), jnp.float32),
        scratch_shapes=[
            pltpu.VMEM((nbuf, blk, D), jnp.float32),
            pltpu.SemaphoreType.DMA((nbuf,)),
        ],
    )(x, w1t, b1r, w2, b2c)


def kernel(geo_id, emb_table, W1, b1, W2, b2):
    nchunks = 1
    nb = B // nchunks
    idx3d = geo_id.astype(jnp.int32).reshape(nchunks, nb // 128, 128)
    w1t = W1.T
    b1r = b1.reshape(1, H)
    b2c = b2.reshape(OUT, 1)
    xs = [_sc_gather(emb_table, idx3d[c], nb) for c in range(nchunks)]
    outs = [_tc_mlp(xs[c], w1t, b1r, W2, b2c, nb) for c in range(nchunks)]
    return jnp.concatenate(outs, axis=1).T


# speedup vs baseline: 1.0351x; 1.0351x over previous
import functools

import jax
import jax.numpy as jnp
from jax import lax
from jax.experimental import pallas as pl
from jax.experimental.pallas import tpu as pltpu
from jax.experimental.pallas import tpu_sc as plsc

B = 16384
D = 128
H = 512
OUT = 3

_info = plsc.get_sparse_core_info()
_NC, _NS = _info.num_cores, _info.num_subcores
_NW = _NC * _NS


def _sc_gather(table, idx2d, nb):
    ch = (nb // 128) // _NW
    mesh = plsc.VectorSubcoreMesh(core_axis_name="c", subcore_axis_name="s")

    @functools.partial(
        pl.kernel,
        mesh=mesh,
        out_type=jax.ShapeDtypeStruct((nb, D), jnp.float32),
        scratch_types=[
            pltpu.VMEM((ch, 128), jnp.int32),
            pltpu.VMEM((ch, 128, D), jnp.float32),
            pltpu.SemaphoreType.DMA,
            pltpu.SemaphoreType.DMA,
        ],
    )
    def k(table_hbm, idx_hbm, out_hbm, idx_v, rows_v, gsem, osem):
        wid = lax.axis_index("s") * _NC + lax.axis_index("c")
        base = wid * ch
        pltpu.sync_copy(idx_hbm.at[pl.ds(base, ch)], idx_v)
        gathers = [
            pltpu.async_copy(table_hbm.at[idx_v.at[j]], rows_v.at[j], gsem)
            for j in range(ch)
        ]
        out_copies = []
        for j in range(ch):
            gathers[j].wait()
            out_copies.append(
                pltpu.async_copy(
                    rows_v.at[j], out_hbm.at[pl.ds((base + j) * 128, 128)], osem
                )
            )
        for c in out_copies:
            c.wait()

    return k(table, idx2d)


def _tc_mlp(x, w1t, b1r, w2, b2c, nb):
    blk = min(4096, nb)

    def body(x_ref, w1_ref, b1_ref, w2_ref, b2_ref, o_ref):
        h = jnp.dot(x_ref[:], w1_ref[:], preferred_element_type=jnp.float32)
        h = jnp.maximum(h + b1_ref[:], 0.0)
        ot = lax.dot_general(
            w2_ref[:], h, (((1,), (1,)), ((), ())),
            preferred_element_type=jnp.float32,
        )
        o_ref[:] = ot + b2_ref[:]

    return pl.pallas_call(
        body,
        grid=(nb // blk,),
        in_specs=[
            pl.BlockSpec((blk, D), lambda i: (i, 0)),
            pl.BlockSpec((D, H), lambda i: (0, 0)),
            pl.BlockSpec((1, H), lambda i: (0, 0)),
            pl.BlockSpec((OUT, H), lambda i: (0, 0)),
            pl.BlockSpec((OUT, 1), lambda i: (0, 0)),
        ],
        out_specs=pl.BlockSpec((OUT, blk), lambda i: (0, i)),
        out_shape=jax.ShapeDtypeStruct((OUT, nb), jnp.float32),
    )(x, w1t, b1r, w2, b2c)


def kernel(geo_id, emb_table, W1, b1, W2, b2):
    idx2d = geo_id.astype(jnp.int32).reshape(B // 128, 128)
    x = _sc_gather(emb_table, idx2d, B)
    out_t = _tc_mlp(x, W1.T, b1.reshape(1, H), W2, b2.reshape(OUT, 1), B)
    return out_t.T
